# Initial kernel scaffold; baseline (speedup 1.0000x reference)
import jax
import jax.numpy as jnp
from jax.experimental import pallas as pl


def kernel(xc_nn_norm, target, pred_m0, pred_m1, pred_m2, pred_m3, W, b):
    raise NotImplementedError("write your pallas kernel here")


# speedup vs baseline: 2.1072x; 2.1072x over previous
import jax
import jax.numpy as jnp
from jax import lax
from jax.experimental import pallas as pl

_T, _T2, _B, _D, _M = 2000, 1635, 1000, 35, 4
_R = 5000
_NBLK = (_T2 * _B) // _R
_OFF = ((_T - _T2) * _B) // _R


def _ens_kernel(xc_ref, p0_ref, p1_ref, p2_ref, p3_ref, w_ref, b_ref, out_ref):
    x = xc_ref[...]
    w = w_ref[...]
    logits = lax.dot_general(
        w, x, (((0,), (1,)), ((), ())),
        preferred_element_type=jnp.float32)
    b = b_ref[...]
    l0 = logits[0:1, :] + b[0, 0]
    l1 = logits[1:2, :] + b[0, 1]
    l2 = logits[2:3, :] + b[0, 2]
    l3 = logits[3:4, :] + b[0, 3]
    p01 = jnp.where(l0 >= l1, p0_ref[0], p1_ref[0])
    v01 = jnp.maximum(l0, l1)
    p23 = jnp.where(l2 >= l3, p2_ref[0], p3_ref[0])
    v23 = jnp.maximum(l2, l3)
    out_ref[0] = jnp.where(v01 >= v23, p01, p23)


def kernel(xc_nn_norm, target, pred_m0, pred_m1, pred_m2, pred_m3, W, b):
    del target
    xc = xc_nn_norm.reshape(_T * _B, _D)
    p0 = pred_m0.reshape(_NBLK, 1, _R)
    p1 = pred_m1.reshape(_NBLK, 1, _R)
    p2 = pred_m2.reshape(_NBLK, 1, _R)
    p3 = pred_m3.reshape(_NBLK, 1, _R)
    pspec = pl.BlockSpec((1, 1, _R), lambda i: (i, 0, 0))
    out = pl.pallas_call(
        _ens_kernel,
        grid=(_NBLK,),
        in_specs=[
            pl.BlockSpec((_R, _D), lambda i: (i + _OFF, 0)),
            pspec, pspec, pspec, pspec,
            pl.BlockSpec((_D, _M), lambda i: (0, 0)),
            pl.BlockSpec((1, _M), lambda i: (0, 0)),
        ],
        out_specs=pl.BlockSpec((1, 1, _R), lambda i: (i, 0, 0)),
        out_shape=jax.ShapeDtypeStruct((_NBLK, 1, _R), jnp.float32),
    )(xc, p0, p1, p2, p3, W, b.reshape(1, _M))
    return out.reshape(_T2, _B)
